# baseline (speedup 1.0000x reference)
import jax
import jax.numpy as jnp
from jax.experimental import pallas as pl
from jax.experimental.pallas import tpu as pltpu

_OFFSET = 16
_CLASSES = 8
_FEAT = _OFFSET * _CLASSES
_K = 64
_MAX_PTS = 1024
_NA = 128
_NV = _NA * _NA
_TOP_ROWS = _MAX_PTS // _NA
_TI = 32
_NTILES = _NA // _TI


def _score_phase(a_ref, bt_ref, wn_ref, bn_ref, wm_ref, bm_ref,
                 nmat_ref, mmat_ref):
    f_iota = jax.lax.broadcasted_iota(jnp.int32, (1, 1, _FEAT), 2)
    l_sh = f_iota >> 3
    c_id = f_iota & 7
    wn = wn_ref[...]
    bn = bn_ref[...]
    wm = wm_ref[...]
    bm = bm_ref[...]
    c0 = bt_ref[0, 0:1, :]
    c1 = bt_ref[0, 1:2, :]
    c2 = bt_ref[0, 2:3, :]

    def tile_body(s, _):
        rows = a_ref[0, pl.ds(s * _TI, _TI), :]
        f0 = rows[:, 0:1] + c0
        f1 = rows[:, 1:2] + c1
        f2 = rows[:, 2:3] + c2
        tok = ((f0[:, :, None] >> l_sh) & 1) \
            + 2 * ((f1[:, :, None] >> l_sh) & 1) \
            + 4 * ((f2[:, :, None] >> l_sh) & 1)
        feat = (tok == c_id).astype(jnp.float32)
        feat2 = feat.reshape(_TI * _NA, _FEAT)
        mm_n = jax.nn.relu(jnp.dot(feat2, wn) + bn)
        nmat_ref[pl.ds(s * _TI, _TI), :] = jnp.sum(
            mm_n.reshape(_TI, _NA, _K) / _K, axis=-1)
        mm_m = jax.nn.relu(jnp.dot(feat2, wm) + bm)
        mmat_ref[pl.ds(s * _TI, _TI), :] = jnp.sum(
            mm_m.reshape(_TI, _NA, _K) / _K, axis=-1)
        return 0

    jax.lax.fori_loop(0, _NTILES, tile_body, 0)


def _sort_stage(flat, v_ref, ix_ref, k, j, r, axis):
    v = v_ref[...]
    ix = ix_ref[...]
    up_mask = (flat & j) != 0
    want_larger = ((flat & k) == 0) == ((flat & j) == 0)
    vu = pltpu.roll(v, r, axis)
    vd = pltpu.roll(v, _NA - r, axis)
    iu = pltpu.roll(ix, r, axis)
    idn = pltpu.roll(ix, _NA - r, axis)
    vp = jnp.where(up_mask, vu, vd)
    ip = jnp.where(up_mask, iu, idn)
    self_lt = (v < vp) | ((v == vp) & (ix > ip))
    take = self_lt == want_larger
    v_ref[...] = jnp.where(take, vp, v)
    ix_ref[...] = jnp.where(take, ip, ix)


def _decoder_body(a_ref, bt_ref, wn_ref, bn_ref, wm_ref,
                  bm_ref, af_ref, vals_ref, sel_ref,
                  nmat_ref, mmat_ref, v_ref, ix_ref):
    _score_phase(a_ref, bt_ref, wn_ref, bn_ref, wm_ref, bm_ref,
                 nmat_ref, mmat_ref)

    row_i = jax.lax.broadcasted_iota(jnp.int32, (_NA, _NA), 0)
    col_i = jax.lax.broadcasted_iota(jnp.int32, (_NA, _NA), 1)
    flat = row_i * _NA + col_i

    v_ref[...] = jax.lax.dot_general(
        nmat_ref[...], mmat_ref[...],
        dimension_numbers=(((1,), (1,)), ((), ())))
    ix_ref[...] = flat

    for m in range(1, 15):
        k = 1 << m
        nrow = max(0, m - 7)
        if nrow > 0:
            def row_body(s, _, m=m, k=k):
                t = (m - 1) - s
                j = jnp.int32(1) << t
                r = jnp.int32(1) << (t - 7)
                _sort_stage(flat, v_ref, ix_ref, k, j, r, 0)
                return 0
            jax.lax.fori_loop(0, nrow, row_body, 0)

        nlane = min(m, 7)
        lane_t0 = min(m - 1, 6)
        def lane_body(s, _, k=k, lane_t0=lane_t0):
            t = lane_t0 - s
            j = jnp.int32(1) << t
            _sort_stage(flat, v_ref, ix_ref, k, j, j, 1)
            return 0
        jax.lax.fori_loop(0, nlane, lane_body, 0)

    v_top = v_ref[0:_TOP_ROWS, :]
    ix_top = ix_ref[0:_TOP_ROWS, :]
    idxmod = jax.lax.rem(ix_top, jnp.int32(384))

    af = af_ref[0]
    tv = jax.lax.broadcasted_iota(jnp.int32, (1, 1, 384), 2)
    selm = jnp.where(idxmod[:, :, None] == tv, af[None, :, :], 0.0)
    sel2 = jnp.sum(selm, axis=-1)

    pos = v_top > 0
    vals_ref[0] = jnp.where(pos, v_top, 0.0)
    sel_ref[0] = jnp.where(pos, sel2, 0.0)


def _make_call(interpret=False):
    bsz = 8
    grid = (bsz,)
    in_specs = [
        pl.BlockSpec((1, _NA, 3), lambda b: (b, 0, 0)),
        pl.BlockSpec((1, 3, _NA), lambda b: (b, 0, 0)),
        pl.BlockSpec((_FEAT, _K), lambda b: (0, 0)),
        pl.BlockSpec((1, _K), lambda b: (0, 0)),
        pl.BlockSpec((_FEAT, _K), lambda b: (0, 0)),
        pl.BlockSpec((1, _K), lambda b: (0, 0)),
        pl.BlockSpec((1, 1, 384), lambda b: (b, 0, 0)),
    ]
    out_specs = [
        pl.BlockSpec((1, _TOP_ROWS, _NA), lambda b: (b, 0, 0)),
        pl.BlockSpec((1, _TOP_ROWS, _NA), lambda b: (b, 0, 0)),
    ]
    out_shape = [
        jax.ShapeDtypeStruct((bsz, _TOP_ROWS, _NA), jnp.float32),
        jax.ShapeDtypeStruct((bsz, _TOP_ROWS, _NA), jnp.float32),
    ]
    scratch_shapes = [
        pltpu.VMEM((_NA, _NA), jnp.float32),
        pltpu.VMEM((_NA, _NA), jnp.float32),
        pltpu.VMEM((_NA, _NA), jnp.float32),
        pltpu.VMEM((_NA, _NA), jnp.int32),
    ]
    return pl.pallas_call(_decoder_body, grid=grid, in_specs=in_specs,
                          out_specs=out_specs, out_shape=out_shape,
                          scratch_shapes=scratch_shapes,
                          interpret=interpret)


def kernel(a, b, W_n, b_n, W_m, b_m):
    bsz = a.shape[0]
    bt = jnp.transpose(b, (0, 2, 1))
    aflat = a.reshape(bsz, 1, 384).astype(jnp.float32)
    call = _make_call()
    vals, sel = call(a, bt, W_n, b_n.reshape(1, _K), W_m,
                     b_m.reshape(1, _K), aflat)
    return vals.reshape(bsz, _MAX_PTS), sel.reshape(bsz, _MAX_PTS)


# speedup vs baseline: 1.1981x; 1.1981x over previous
import jax
import jax.numpy as jnp
from jax.experimental import pallas as pl
from jax.experimental.pallas import tpu as pltpu

_OFFSET = 16
_CLASSES = 8
_FEAT = _OFFSET * _CLASSES
_K = 64
_MAX_PTS = 1024
_NA = 128
_NV = _NA * _NA
_TOP_ROWS = _MAX_PTS // _NA
_TI = 32
_NTILES = _NA // _TI


def _score_phase(a_ref, bt_ref, wn_ref, bn_ref, wm_ref, bm_ref,
                 nmat_ref, mmat_ref):
    f_iota = jax.lax.broadcasted_iota(jnp.int32, (1, 1, _FEAT), 2)
    l_sh = f_iota >> 3
    c_id = f_iota & 7
    wn = wn_ref[...]
    bn = bn_ref[...]
    wm = wm_ref[...]
    bm = bm_ref[...]
    c0 = bt_ref[0, 0:1, :]
    c1 = bt_ref[0, 1:2, :]
    c2 = bt_ref[0, 2:3, :]

    def tile_body(s, _):
        rows = a_ref[0, pl.ds(s * _TI, _TI), :]
        f0 = rows[:, 0:1] + c0
        f1 = rows[:, 1:2] + c1
        f2 = rows[:, 2:3] + c2
        tok = ((f0[:, :, None] >> l_sh) & 1) \
            + 2 * ((f1[:, :, None] >> l_sh) & 1) \
            + 4 * ((f2[:, :, None] >> l_sh) & 1)
        feat = (tok == c_id).astype(jnp.float32)
        feat2 = feat.reshape(_TI * _NA, _FEAT)
        mm_n = jax.nn.relu(jnp.dot(feat2, wn) + bn)
        nmat_ref[pl.ds(s * _TI, _TI), :] = jnp.sum(
            mm_n.reshape(_TI, _NA, _K) / _K, axis=-1)
        mm_m = jax.nn.relu(jnp.dot(feat2, wm) + bm)
        mmat_ref[pl.ds(s * _TI, _TI), :] = jnp.sum(
            mm_m.reshape(_TI, _NA, _K) / _K, axis=-1)
        return 0

    jax.lax.fori_loop(0, _NTILES, tile_body, 0)


def _static_roll(x, dist, rows):
    if dist < _NA:
        return jnp.concatenate([x[:, dist:], x[:, :dist]], axis=1)
    r = dist // _NA
    return jnp.concatenate([x[r:, :], x[:r, :]], axis=0)


def _static_stage(v, ix, flat, k, j, rows):
    up_mask = (flat & j) != 0
    want_larger = ((flat & k) == 0) == ((flat & j) == 0)
    nel = rows * _NA
    vd = _static_roll(v, j, rows)
    vu = _static_roll(v, nel - j if j >= _NA else _NA - j, rows)
    idn = _static_roll(ix, j, rows)
    iu = _static_roll(ix, nel - j if j >= _NA else _NA - j, rows)
    vp = jnp.where(up_mask, vu, vd)
    ip = jnp.where(up_mask, iu, idn)
    self_lt = (v < vp) | ((v == vp) & (ix > ip))
    take = self_lt == want_larger
    return jnp.where(take, vp, v), jnp.where(take, ip, ix)


def _sort_stage(flat, v_ref, ix_ref, k, j, r, axis):
    v = v_ref[...]
    ix = ix_ref[...]
    up_mask = (flat & j) != 0
    want_larger = ((flat & k) == 0) == ((flat & j) == 0)
    vu = pltpu.roll(v, r, axis)
    vd = pltpu.roll(v, _NA - r, axis)
    iu = pltpu.roll(ix, r, axis)
    idn = pltpu.roll(ix, _NA - r, axis)
    vp = jnp.where(up_mask, vu, vd)
    ip = jnp.where(up_mask, iu, idn)
    self_lt = (v < vp) | ((v == vp) & (ix > ip))
    take = self_lt == want_larger
    v_ref[...] = jnp.where(take, vp, v)
    ix_ref[...] = jnp.where(take, ip, ix)


def _decoder_body(a_ref, bt_ref, wn_ref, bn_ref, wm_ref,
                  bm_ref, af_ref, vals_ref, sel_ref,
                  nmat_ref, mmat_ref, v_ref, ix_ref):
    _score_phase(a_ref, bt_ref, wn_ref, bn_ref, wm_ref, bm_ref,
                 nmat_ref, mmat_ref)

    row_i = jax.lax.broadcasted_iota(jnp.int32, (_NA, _NA), 0)
    col_i = jax.lax.broadcasted_iota(jnp.int32, (_NA, _NA), 1)
    flat = row_i * _NA + col_i

    v_ref[...] = jax.lax.dot_general(
        nmat_ref[...], mmat_ref[...],
        dimension_numbers=(((1,), (1,)), ((), ())))
    ix_ref[...] = flat

    for m in range(1, 11):
        k = 1 << m
        nrow = max(0, m - 7)
        if nrow > 0:
            def row_body(s, _, m=m, k=k):
                t = (m - 1) - s
                j = jnp.int32(1) << t
                r = jnp.int32(1) << (t - 7)
                _sort_stage(flat, v_ref, ix_ref, k, j, r, 0)
                return 0
            jax.lax.fori_loop(0, nrow, row_body, 0)

        nlane = min(m, 7)
        lane_t0 = min(m - 1, 6)
        def lane_body(s, _, k=k, lane_t0=lane_t0):
            t = lane_t0 - s
            j = jnp.int32(1) << t
            _sort_stage(flat, v_ref, ix_ref, k, j, j, 1)
            return 0
        jax.lax.fori_loop(0, nlane, lane_body, 0)

    v = v_ref[...]
    ix = ix_ref[...]
    rows = _NA
    while rows > _TOP_ROWS:
        g = rows // 16
        v4 = v.reshape(g, 16, _NA)
        i4 = ix.reshape(g, 16, _NA)
        av, bv = v4[:, :8, :], v4[:, 8:, :]
        ai, bi = i4[:, :8, :], i4[:, 8:, :]
        lt = (av < bv) | ((av == bv) & (ai > bi))
        rows = rows // 2
        v = jnp.where(lt, bv, av).reshape(rows, _NA)
        ix = jnp.where(lt, bi, ai).reshape(rows, _NA)
        kk = 1024 if rows > _TOP_ROWS else 2048
        fl = jax.lax.broadcasted_iota(jnp.int32, (rows, _NA), 0) * _NA \
            + jax.lax.broadcasted_iota(jnp.int32, (rows, _NA), 1)
        j = 512
        while j >= 1:
            v, ix = _static_stage(v, ix, fl, kk, j, rows)
            j //= 2

    v_top = v
    ix_top = ix
    idxmod = jax.lax.rem(ix_top, jnp.int32(384))

    af = af_ref[0]
    tv = jax.lax.broadcasted_iota(jnp.int32, (1, 1, 384), 2)
    selm = jnp.where(idxmod[:, :, None] == tv, af[None, :, :], 0.0)
    sel2 = jnp.sum(selm, axis=-1)

    pos = v_top > 0
    vals_ref[0] = jnp.where(pos, v_top, 0.0)
    sel_ref[0] = jnp.where(pos, sel2, 0.0)


def _make_call(interpret=False):
    bsz = 8
    grid = (bsz,)
    in_specs = [
        pl.BlockSpec((1, _NA, 3), lambda b: (b, 0, 0)),
        pl.BlockSpec((1, 3, _NA), lambda b: (b, 0, 0)),
        pl.BlockSpec((_FEAT, _K), lambda b: (0, 0)),
        pl.BlockSpec((1, _K), lambda b: (0, 0)),
        pl.BlockSpec((_FEAT, _K), lambda b: (0, 0)),
        pl.BlockSpec((1, _K), lambda b: (0, 0)),
        pl.BlockSpec((1, 1, 384), lambda b: (b, 0, 0)),
    ]
    out_specs = [
        pl.BlockSpec((1, _TOP_ROWS, _NA), lambda b: (b, 0, 0)),
        pl.BlockSpec((1, _TOP_ROWS, _NA), lambda b: (b, 0, 0)),
    ]
    out_shape = [
        jax.ShapeDtypeStruct((bsz, _TOP_ROWS, _NA), jnp.float32),
        jax.ShapeDtypeStruct((bsz, _TOP_ROWS, _NA), jnp.float32),
    ]
    scratch_shapes = [
        pltpu.VMEM((_NA, _NA), jnp.float32),
        pltpu.VMEM((_NA, _NA), jnp.float32),
        pltpu.VMEM((_NA, _NA), jnp.float32),
        pltpu.VMEM((_NA, _NA), jnp.int32),
    ]
    return pl.pallas_call(_decoder_body, grid=grid, in_specs=in_specs,
                          out_specs=out_specs, out_shape=out_shape,
                          scratch_shapes=scratch_shapes,
                          interpret=interpret)


def kernel(a, b, W_n, b_n, W_m, b_m):
    bsz = a.shape[0]
    bt = jnp.transpose(b, (0, 2, 1))
    aflat = a.reshape(bsz, 1, 384).astype(jnp.float32)
    call = _make_call()
    vals, sel = call(a, bt, W_n, b_n.reshape(1, _K), W_m,
                     b_m.reshape(1, _K), aflat)
    return vals.reshape(bsz, _MAX_PTS), sel.reshape(bsz, _MAX_PTS)
